# baseline (speedup 1.0000x reference)
import functools

import jax
import jax.numpy as jnp
from jax import lax
from jax.experimental import pallas as pl
from jax.experimental.pallas import tpu as pltpu
from jax.experimental.pallas import tpu_sc as plsc

NC = 2
NS = 16
L = 16
NW = NC * NS
CHUNK = 128
PK = 128 // L


def _encode(nodes, w_e16, b_e16):
  n, d = nodes.shape
  blk = 1000
  grid = n // blk

  def body(x_ref, w_ref, b_ref, o_ref):
    acc = jnp.dot(x_ref[...], w_ref[...], preferred_element_type=jnp.float32)
    o_ref[...] = jnp.maximum(acc + b_ref[...], 0.0)

  return pl.pallas_call(
      body,
      grid=(grid,),
      in_specs=[
          pl.BlockSpec((blk, d), lambda i: (i, 0)),
          pl.BlockSpec((d, L), lambda i: (0, 0)),
          pl.BlockSpec((1, L), lambda i: (0, 0)),
      ],
      out_specs=pl.BlockSpec((blk, L), lambda i: (i, 0)),
      out_shape=jax.ShapeDtypeStruct((n, L), jnp.float32),
  )(nodes, w_e16, b_e16)


def _sc_segment_sum(vodes16, senders_p, receivers_p, n_acc, n_chunks):
  rows_per_tile = n_acc // NS
  n_vod = vodes16.shape[0]
  mesh = plsc.VectorSubcoreMesh(core_axis_name="c", subcore_axis_name="s")

  prows_per_tile = rows_per_tile // PK

  @functools.partial(
      pl.kernel,
      out_type=(jax.ShapeDtypeStruct((NC, n_acc // PK, PK, L), jnp.float32),
                jax.ShapeDtypeStruct((n_acc // PK, PK, L), jnp.float32)),
      mesh=mesh,
      scratch_types=[
          pltpu.VMEM((n_chunks, CHUNK), jnp.int32),
          pltpu.VMEM((n_chunks, CHUNK), jnp.int32),
          pltpu.VMEM((CHUNK, L), jnp.float32),
          pltpu.VMEM((CHUNK, L), jnp.float32),
          pltpu.VMEM((rows_per_tile, L), jnp.float32),
          pltpu.VMEM((prows_per_tile, PK, L), jnp.float32),
          pltpu.VMEM_SHARED((n_acc, L), jnp.float32),
          pltpu.VMEM_SHARED((n_acc, L), jnp.float32),
          pltpu.SemaphoreType.DMA,
          pltpu.SemaphoreType.DMA,
          pltpu.SemaphoreType.DMA,
          pltpu.SemaphoreType.DMA,
      ],
      compiler_params=pltpu.CompilerParams(use_tc_tiling_on_sc=False),
  )
  def k(vodes_hbm, s_hbm, r_hbm, out_hbm, vpk_hbm, idx_s, idx_r, rows0, rows1,
        stage, pkbuf, vod_sh, acc, sem0, sem1, semi, semo):
    c = lax.axis_index("c")
    s = lax.axis_index("s")
    wid = s * NC + c
    bufs = (rows0, rows1)
    sems = (sem0, sem1)

    row0 = wid * n_chunks
    s_idx = pltpu.async_copy(s_hbm.at[pl.ds(row0, n_chunks)], idx_s, semi)
    r_idx = pltpu.async_copy(r_hbm.at[pl.ds(row0, n_chunks)], idx_r, semi)
    v0 = s * rows_per_tile
    last_rows = n_vod - (NS - 1) * rows_per_tile

    @pl.when(s < NS - 1)
    def _():
      pltpu.async_copy(vodes_hbm.at[pl.ds(v0, rows_per_tile)],
                       vod_sh.at[pl.ds(v0, rows_per_tile)], semi)

    @pl.when(s == NS - 1)
    def _():
      pltpu.async_copy(vodes_hbm.at[pl.ds(v0, last_rows)],
                       vod_sh.at[pl.ds(v0, last_rows)], semi)

    def zero_body(i, carry):
      stage[i, :] = jnp.zeros((L,), jnp.float32)
      return carry

    lax.fori_loop(0, rows_per_tile, zero_body, 0)
    r0 = s * rows_per_tile
    pltpu.sync_copy(stage, acc.at[pl.ds(r0, rows_per_tile)])
    s_idx.wait()
    r_idx.wait()

    @pl.when(s < NS - 1)
    def _():
      pltpu.make_async_copy(vodes_hbm.at[pl.ds(0, rows_per_tile)],
                            vod_sh.at[pl.ds(0, rows_per_tile)], semi).wait()

    @pl.when(s == NS - 1)
    def _():
      pltpu.make_async_copy(vodes_hbm.at[pl.ds(0, last_rows)],
                            vod_sh.at[pl.ds(0, last_rows)], semi).wait()

    plsc.subcore_barrier()

    pltpu.async_copy(vod_sh.at[idx_s.at[0]], bufs[0], sems[0])

    @pl.loop(0, n_chunks, step=2)
    def edge_loop(io):
      for b in range(2):
        i = io + b
        nxt = bufs[1 - b]

        @pl.when(i + 1 < n_chunks)
        def _():
          pltpu.async_copy(vod_sh.at[idx_s.at[i + 1]], nxt, sems[1 - b])

        pltpu.make_async_copy(vod_sh.at[idx_s.at[i]], bufs[b],
                              sems[b]).wait()
        pltpu.sync_copy(bufs[b], acc.at[idx_r.at[i]], add=True)

    plsc.subcore_barrier()

    p0 = s * prows_per_tile

    @pl.loop(0, prows_per_tile)
    def fire_msg(q):
      pltpu.async_copy(acc.at[pl.ds(r0 + PK * q, PK)], pkbuf.at[q], semo)

    @pl.loop(0, prows_per_tile)
    def drain_msg(q):
      pltpu.make_async_copy(acc.at[pl.ds(r0 + PK * q, PK)], pkbuf.at[q],
                            semo).wait()

    pltpu.sync_copy(pkbuf, out_hbm.at[c].at[pl.ds(p0, prows_per_tile)])

    @pl.when(c == 0)
    def _():
      @pl.loop(0, prows_per_tile)
      def fire_v(q):
        pltpu.async_copy(vod_sh.at[pl.ds(r0 + PK * q, PK)], pkbuf.at[q], semo)

      @pl.loop(0, prows_per_tile)
      def drain_v(q):
        pltpu.make_async_copy(vod_sh.at[pl.ds(r0 + PK * q, PK)], pkbuf.at[q],
                              semo).wait()

      pltpu.sync_copy(pkbuf, vpk_hbm.at[pl.ds(p0, prows_per_tile)])

  return k(vodes16, senders_p, receivers_p)


def _finalize(v_pk, msg_pk, np_rows, wi_tiled, b_i, w_a1p, b_a1, w_a2, b_a2,
              w_c1p, b_c1, w_c2, b_c2):
  def body(v_ref, m_ref, wi_ref, bi_ref, wa1_ref, ba1_ref, wa2_ref, ba2_ref,
           wc1_ref, bc1_ref, wc2_ref, bc2_ref, lo_ref, vo_ref):
    v = v_ref[:np_rows, :]
    m = m_ref[0, :np_rows, :] + m_ref[1, :np_rows, :]
    wi_t = wi_ref[...]
    bi = bi_ref[0, 0]
    e_t = (lax.broadcasted_iota(jnp.int32, (PK * L, PK), 0) // L ==
           lax.broadcasted_iota(jnp.int32, (PK * L, PK), 1)
           ).astype(jnp.float32)
    e = (lax.broadcasted_iota(jnp.int32, (PK, PK * L), 0) ==
         lax.broadcasted_iota(jnp.int32, (PK, PK * L), 1) // L
         ).astype(jnp.float32)
    f_sel = (lax.broadcasted_iota(jnp.int32, (PK * L, L), 0) % L ==
             lax.broadcasted_iota(jnp.int32, (PK * L, L), 1)
             ).astype(jnp.float32)

    a = 10.0 * (jnp.dot(v * wi_t, e_t,
                        preferred_element_type=jnp.float32) + bi)
    a_b = jnp.dot(a, e, preferred_element_type=jnp.float32)
    vnew = v + a_b * m
    infl = jnp.dot(vnew * wi_t, e_t,
                   preferred_element_type=jnp.float32) + bi
    mx = jnp.max(infl)
    w = jnp.exp(infl - mx)
    denom = jnp.sum(w)
    w_b = jnp.dot(w, e, preferred_element_type=jnp.float32)
    pooled = jnp.sum(vnew * w_b, axis=0, keepdims=True) / denom
    gr = jnp.dot(pooled, f_sel, preferred_element_type=jnp.float32)
    h_a = jnp.maximum(
        jnp.dot(gr, wa1_ref[...], preferred_element_type=jnp.float32)
        + ba1_ref[...], 0.0)
    lo_ref[...] = (jnp.dot(h_a, wa2_ref[...],
                           preferred_element_type=jnp.float32) + ba2_ref[...])
    h_c = jnp.maximum(
        jnp.dot(gr, wc1_ref[...], preferred_element_type=jnp.float32)
        + bc1_ref[...], 0.0)
    vo_ref[...] = (jnp.dot(h_c, wc2_ref[...],
                           preferred_element_type=jnp.float32) + bc2_ref[...])

  return pl.pallas_call(
      body,
      out_shape=(jax.ShapeDtypeStruct((1, 3), jnp.float32),
                 jax.ShapeDtypeStruct((1, 1), jnp.float32)),
  )(v_pk, msg_pk, wi_tiled, b_i.reshape(1, 1), w_a1p, b_a1.reshape(1, -1),
    w_a2, b_a2.reshape(1, -1), w_c1p, b_c1.reshape(1, -1),
    w_c2, b_c2.reshape(1, -1))


def kernel(nodes, senders, receivers, W_e, b_e, W_i, b_i,
           W_a1, b_a1, W_a2, b_a2, W_c1, b_c1, W_c2, b_c2):
  n, d = nodes.shape
  e = senders.shape[0]
  f = W_e.shape[1]

  w_e16 = jnp.pad(W_e, ((0, 0), (0, L - f)))
  b_e16 = jnp.pad(b_e, (0, L - f)).reshape(1, L)
  wi_tiled = jnp.tile(jnp.pad(W_i[:, 0], (0, L - f)), PK).reshape(1, PK * L)
  w_a1p = jnp.pad(W_a1, ((0, L - f), (0, 0)))
  w_c1p = jnp.pad(W_c1, ((0, L - f), (0, 0)))

  vodes16 = _encode(nodes, w_e16, b_e16)

  epw = -(-e // (NW * 2 * CHUNK)) * 2 * CHUNK
  e_pad = epw * NW
  n_chunks = epw // CHUNK
  pad = e_pad - e
  senders_p = jnp.concatenate(
      [senders, jnp.zeros((pad,), jnp.int32)]).reshape(NW * n_chunks, CHUNK)
  receivers_p = jnp.concatenate(
      [receivers, jnp.full((pad,), n, jnp.int32)]).reshape(NW * n_chunks, CHUNK)
  n_acc = -(-(n + 1) // (NS * 8 * PK)) * NS * 8 * PK

  msg, v_pk4 = _sc_segment_sum(vodes16, senders_p, receivers_p, n_acc,
                               n_chunks)
  msg_pk = msg.reshape(NC, n_acc // PK, PK * L)
  v_pk = v_pk4.reshape(n_acc // PK, PK * L)

  logits, value = _finalize(v_pk, msg_pk, n // PK, wi_tiled, b_i, w_a1p,
                            b_a1, W_a2, b_a2, w_c1p, b_c1, W_c2, b_c2)
  return (logits[0], value[0])


# speedup vs baseline: 1.1387x; 1.1387x over previous
import functools

import jax
import jax.numpy as jnp
from jax import lax
from jax.experimental import pallas as pl
from jax.experimental.pallas import tpu as pltpu
from jax.experimental.pallas import tpu_sc as plsc

NC = 2
NS = 16
L = 16
NW = NC * NS
CHUNK = 128


def _encode(nodes, w_e16, b_e16):
  n, d = nodes.shape
  blk = 1000
  grid = n // blk

  def body(x_ref, w_ref, b_ref, o_ref):
    acc = jnp.dot(x_ref[...], w_ref[...], preferred_element_type=jnp.float32)
    o_ref[...] = jnp.maximum(acc + b_ref[...], 0.0)

  return pl.pallas_call(
      body,
      grid=(grid,),
      in_specs=[
          pl.BlockSpec((blk, d), lambda i: (i, 0)),
          pl.BlockSpec((d, L), lambda i: (0, 0)),
          pl.BlockSpec((1, L), lambda i: (0, 0)),
      ],
      out_specs=pl.BlockSpec((blk, L), lambda i: (i, 0)),
      out_shape=jax.ShapeDtypeStruct((n, L), jnp.float32),
  )(nodes, w_e16, b_e16)


def _sc_segment_sum(vodes16, senders_p, receivers_p, n_acc, n_chunks):
  rows_per_tile = n_acc // NS
  n_vod = vodes16.shape[0]
  vrows_per_tile = n_vod // NS
  mesh = plsc.VectorSubcoreMesh(core_axis_name="c", subcore_axis_name="s")

  @functools.partial(
      pl.kernel,
      out_type=jax.ShapeDtypeStruct((NC, n_acc, L), jnp.float32),
      mesh=mesh,
      scratch_types=[
          pltpu.VMEM((n_chunks, CHUNK), jnp.int32),
          pltpu.VMEM((n_chunks, CHUNK), jnp.int32),
          pltpu.VMEM((CHUNK, L), jnp.float32),
          pltpu.VMEM((CHUNK, L), jnp.float32),
          pltpu.VMEM((rows_per_tile, L), jnp.float32),
          pltpu.VMEM_SHARED((n_vod, L), jnp.float32),
          pltpu.VMEM_SHARED((n_acc, L), jnp.float32),
          pltpu.SemaphoreType.DMA,
          pltpu.SemaphoreType.DMA,
          pltpu.SemaphoreType.DMA,
      ],
      compiler_params=pltpu.CompilerParams(use_tc_tiling_on_sc=False),
  )
  def k(vodes_hbm, s_hbm, r_hbm, out_hbm, idx_s, idx_r, rows0, rows1, stage,
        vod_sh, acc, sem0, sem1, semi):
    c = lax.axis_index("c")
    s = lax.axis_index("s")
    wid = s * NC + c
    bufs = (rows0, rows1)
    sems = (sem0, sem1)

    row0 = wid * n_chunks
    s_idx = pltpu.async_copy(s_hbm.at[pl.ds(row0, n_chunks)], idx_s, semi)
    r_idx = pltpu.async_copy(r_hbm.at[pl.ds(row0, n_chunks)], idx_r, semi)
    v0 = s * vrows_per_tile
    v_stage = pltpu.async_copy(vodes_hbm.at[pl.ds(v0, vrows_per_tile)],
                               vod_sh.at[pl.ds(v0, vrows_per_tile)], semi)

    def zero_body(i, carry):
      stage[i, :] = jnp.zeros((L,), jnp.float32)
      return carry

    lax.fori_loop(0, rows_per_tile, zero_body, 0)
    r0 = s * rows_per_tile
    pltpu.sync_copy(stage, acc.at[pl.ds(r0, rows_per_tile)])
    s_idx.wait()
    r_idx.wait()
    v_stage.wait()
    plsc.subcore_barrier()

    pltpu.async_copy(vod_sh.at[idx_s.at[0]], bufs[0], sems[0])

    @pl.loop(0, n_chunks, step=2)
    def edge_loop(io):
      for b in range(2):
        i = io + b
        nxt = bufs[1 - b]

        @pl.when(i + 1 < n_chunks)
        def _():
          pltpu.async_copy(vod_sh.at[idx_s.at[i + 1]], nxt, sems[1 - b])

        pltpu.make_async_copy(vod_sh.at[idx_s.at[i]], bufs[b],
                              sems[b]).wait()
        pltpu.sync_copy(bufs[b], acc.at[idx_r.at[i]], add=True)

    plsc.subcore_barrier()

    pltpu.sync_copy(acc.at[pl.ds(r0, rows_per_tile)], stage)
    pltpu.sync_copy(stage, out_hbm.at[c].at[pl.ds(r0, rows_per_tile)])

  return k(vodes16, senders_p, receivers_p)


def _finalize(vodes16, msg, w_i16, b_i, w_a1p, b_a1, w_a2, b_a2,
              w_c1p, b_c1, w_c2, b_c2):
  n = vodes16.shape[0]

  def body(v_ref, m_ref, wi_ref, bi_ref, wa1_ref, ba1_ref, wa2_ref, ba2_ref,
           wc1_ref, bc1_ref, wc2_ref, bc2_ref, lo_ref, vo_ref):
    v = v_ref[...]
    m = m_ref[0, :n, :] + m_ref[1, :n, :]
    wi = wi_ref[...]
    bi = bi_ref[0, 0]
    a = 10.0 * (jnp.dot(v, wi, preferred_element_type=jnp.float32) + bi)
    vnew = v + a * m
    infl = jnp.dot(vnew, wi, preferred_element_type=jnp.float32) + bi
    mx = jnp.max(infl)
    w = jnp.exp(infl - mx)
    denom = jnp.sum(w)
    gr = jnp.sum(vnew * w, axis=0, keepdims=True) / denom
    h_a = jnp.maximum(
        jnp.dot(gr, wa1_ref[...], preferred_element_type=jnp.float32)
        + ba1_ref[...], 0.0)
    lo_ref[...] = (jnp.dot(h_a, wa2_ref[...],
                           preferred_element_type=jnp.float32) + ba2_ref[...])
    h_c = jnp.maximum(
        jnp.dot(gr, wc1_ref[...], preferred_element_type=jnp.float32)
        + bc1_ref[...], 0.0)
    vo_ref[...] = (jnp.dot(h_c, wc2_ref[...],
                           preferred_element_type=jnp.float32) + bc2_ref[...])

  return pl.pallas_call(
      body,
      out_shape=(jax.ShapeDtypeStruct((1, 3), jnp.float32),
                 jax.ShapeDtypeStruct((1, 1), jnp.float32)),
  )(vodes16, msg, w_i16, b_i.reshape(1, 1), w_a1p, b_a1.reshape(1, -1),
    w_a2, b_a2.reshape(1, -1), w_c1p, b_c1.reshape(1, -1),
    w_c2, b_c2.reshape(1, -1))


def kernel(nodes, senders, receivers, W_e, b_e, W_i, b_i,
           W_a1, b_a1, W_a2, b_a2, W_c1, b_c1, W_c2, b_c2):
  n, d = nodes.shape
  e = senders.shape[0]
  f = W_e.shape[1]

  w_e16 = jnp.pad(W_e, ((0, 0), (0, L - f)))
  b_e16 = jnp.pad(b_e, (0, L - f)).reshape(1, L)
  w_i16 = jnp.pad(W_i, ((0, L - f), (0, 0)))
  w_a1p = jnp.pad(W_a1, ((0, L - f), (0, 0)))
  w_c1p = jnp.pad(W_c1, ((0, L - f), (0, 0)))

  vodes16 = _encode(nodes, w_e16, b_e16)

  epw = -(-e // (NW * 2 * CHUNK)) * 2 * CHUNK
  e_pad = epw * NW
  n_chunks = epw // CHUNK
  pad = e_pad - e
  senders_p = jnp.concatenate(
      [senders, jnp.zeros((pad,), jnp.int32)]).reshape(NW * n_chunks, CHUNK)
  receivers_p = jnp.concatenate(
      [receivers, jnp.full((pad,), n, jnp.int32)]).reshape(NW * n_chunks, CHUNK)
  n_acc = -(-(n + 1) // (NS * 8)) * NS * 8

  msg = _sc_segment_sum(vodes16, senders_p, receivers_p, n_acc, n_chunks)

  logits, value = _finalize(vodes16, msg, w_i16, b_i, w_a1p, b_a1, W_a2, b_a2,
                            w_c1p, b_c1, W_c2, b_c2)
  return (logits[0], value[0])
